# Initial kernel scaffold; baseline (speedup 1.0000x reference)
import jax
import jax.numpy as jnp
from jax.experimental import pallas as pl


def kernel(feat, feat_a, adj, graph_neigh, W1, W2, disc_W, disc_b):
    raise NotImplementedError("write your pallas kernel here")


# speedup vs baseline: 1.6403x; 1.6403x over previous
import jax
import jax.numpy as jnp
from jax.experimental import pallas as pl

_N = 4096
_BLK = 512
_GRID = _N // _BLK


def _prep_body(adj_ref, dinvb_ref, adj8_ref):
    i = pl.program_id(0)
    a = adj_ref[...]
    row_ids = jax.lax.broadcasted_iota(jnp.int32, (_BLK, _N), 0) + i * _BLK
    col_ids = jax.lax.broadcasted_iota(jnp.int32, (_BLK, _N), 1)
    ondiag = row_ids == col_ids
    a_sl = jnp.where(ondiag & (a == 0.0), 1.0, a)
    deg = jnp.sum(a_sl, axis=1)
    dinv = jnp.where(deg > 0.0, jax.lax.rsqrt(deg), 0.0)
    dinvb_ref[...] = jnp.broadcast_to(dinv[:, None], (_BLK, 128))
    adj8_ref[...] = a_sl.astype(jnp.int8)


def _xw_body(feat_ref, feata_ref, w1_ref, dinvb_ref, xs_ref):
    xw = jnp.dot(feat_ref[...], w1_ref[...], preferred_element_type=jnp.float32)
    xwa = jnp.dot(feata_ref[...], w1_ref[...], preferred_element_type=jnp.float32)
    xs = jnp.concatenate([xw, xwa], axis=1) * dinvb_ref[...]
    xs_ref[...] = xs.astype(jnp.bfloat16)


def _prop1_body(adj8_ref, xs_ref, dinvb_ref, w2_ref,
                z_ref, emb_ref, emba_ref, embcat_ref, ys_ref):
    a = adj8_ref[...].astype(jnp.bfloat16)
    acc = jnp.dot(a, xs_ref[...], preferred_element_type=jnp.float32)
    zc = acc * dinvb_ref[...]
    z = zc[:, :64]
    za = zc[:, 64:]
    emb = jnp.maximum(z, 0.0)
    emba = jnp.maximum(za, 0.0)
    z_ref[...] = z
    emb_ref[...] = emb
    emba_ref[...] = emba
    embcat_ref[...] = jnp.concatenate([emb, emba], axis=1).astype(jnp.bfloat16)
    ys = jnp.dot(z, w2_ref[...], preferred_element_type=jnp.float32) * dinvb_ref[...]
    ys_ref[...] = ys.astype(jnp.bfloat16)


def _prop2_body(adj8_ref, ys_ref, dinvb_ref, h_ref):
    a = adj8_ref[...].astype(jnp.bfloat16)
    h_ref[...] = jnp.dot(a, ys_ref[...],
                         preferred_element_type=jnp.float32) * dinvb_ref[...]


def _readout_body(gn_ref, embcat_ref, emb_ref, emba_ref, w0_ref, b_ref,
                  ret_ref, reta_ref):
    g = gn_ref[...]
    vs = jnp.dot(g.astype(jnp.bfloat16), embcat_ref[...],
                 preferred_element_type=jnp.float32)
    rs = jnp.sum(g, axis=1)
    gc = vs / rs[:, None]
    gp = gc[:, :64]
    gpa = gc[:, 64:]

    def l2sig(x):
        nrm = jnp.sqrt(jnp.sum(x * x, axis=1, keepdims=True))
        return jax.nn.sigmoid(x / jnp.maximum(nrm, 1e-12))

    gp = l2sig(gp)
    gpa = l2sig(gpa)
    w0 = w0_ref[0]
    hw = jnp.dot(emb_ref[...], w0, preferred_element_type=jnp.float32)
    hwa = jnp.dot(emba_ref[...], w0, preferred_element_type=jnp.float32)
    b = b_ref[0, 0]
    r0 = jnp.sum(hw * gp, axis=1, keepdims=True) + b
    r1 = jnp.sum(hwa * gp, axis=1, keepdims=True) + b
    ra0 = jnp.sum(hwa * gpa, axis=1, keepdims=True) + b
    ra1 = jnp.sum(hw * gpa, axis=1, keepdims=True) + b
    ret_ref[...] = jnp.concatenate([r0, r1], axis=1)
    reta_ref[...] = jnp.concatenate([ra0, ra1], axis=1)


def _row_spec():
    return pl.BlockSpec((_BLK, _N), lambda i: (i, 0))


def _full(shape):
    nd = len(shape)
    return pl.BlockSpec(shape, lambda *_, _nd=nd: (0,) * _nd)


def kernel(feat, feat_a, adj, graph_neigh, W1, W2, disc_W, disc_b):
    f32 = jnp.float32
    bf16 = jnp.bfloat16

    dinvb, adj8 = pl.pallas_call(
        _prep_body,
        grid=(_GRID,),
        in_specs=[_row_spec()],
        out_specs=[pl.BlockSpec((_BLK, 128), lambda i: (i, 0)), _row_spec()],
        out_shape=[jax.ShapeDtypeStruct((_N, 128), f32),
                   jax.ShapeDtypeStruct((_N, _N), jnp.int8)],
    )(adj)

    xs = pl.pallas_call(
        _xw_body,
        in_specs=[_full((_N, 128)), _full((_N, 128)), _full((128, 64)),
                  _full((_N, 128))],
        out_specs=_full((_N, 128)),
        out_shape=jax.ShapeDtypeStruct((_N, 128), bf16),
    )(feat, feat_a, W1, dinvb)

    z, emb, emb_a, embcat, ys = pl.pallas_call(
        _prop1_body,
        grid=(_GRID,),
        in_specs=[_row_spec(), _full((_N, 128)),
                  pl.BlockSpec((_BLK, 128), lambda i: (i, 0)),
                  _full((64, 128))],
        out_specs=[pl.BlockSpec((_BLK, 64), lambda i: (i, 0)),
                   pl.BlockSpec((_BLK, 64), lambda i: (i, 0)),
                   pl.BlockSpec((_BLK, 64), lambda i: (i, 0)),
                   pl.BlockSpec((_BLK, 128), lambda i: (i, 0)),
                   pl.BlockSpec((_BLK, 128), lambda i: (i, 0))],
        out_shape=[jax.ShapeDtypeStruct((_N, 64), f32),
                   jax.ShapeDtypeStruct((_N, 64), f32),
                   jax.ShapeDtypeStruct((_N, 64), f32),
                   jax.ShapeDtypeStruct((_N, 128), bf16),
                   jax.ShapeDtypeStruct((_N, 128), bf16)],
    )(adj8, xs, dinvb, W2)

    h = pl.pallas_call(
        _prop2_body,
        grid=(_GRID,),
        in_specs=[_row_spec(), _full((_N, 128)),
                  pl.BlockSpec((_BLK, 128), lambda i: (i, 0))],
        out_specs=pl.BlockSpec((_BLK, 128), lambda i: (i, 0)),
        out_shape=jax.ShapeDtypeStruct((_N, 128), f32),
    )(adj8, ys, dinvb)

    ret, ret_a = pl.pallas_call(
        _readout_body,
        grid=(_GRID,),
        in_specs=[_row_spec(), _full((_N, 128)),
                  pl.BlockSpec((_BLK, 64), lambda i: (i, 0)),
                  pl.BlockSpec((_BLK, 64), lambda i: (i, 0)),
                  _full((1, 64, 64)), _full((1, 1))],
        out_specs=[pl.BlockSpec((_BLK, 2), lambda i: (i, 0)),
                   pl.BlockSpec((_BLK, 2), lambda i: (i, 0))],
        out_shape=[jax.ShapeDtypeStruct((_N, 2), f32),
                   jax.ShapeDtypeStruct((_N, 2), f32)],
    )(graph_neigh, embcat, emb, emb_a, disc_W, disc_b.reshape(1, 1))

    return (z, h, ret, ret_a, emb, emb_a)


# speedup vs baseline: 1.7655x; 1.0763x over previous
import jax
import jax.numpy as jnp
from jax.experimental import pallas as pl
from jax.experimental.pallas import tpu as pltpu

_N = 4096
_BLK = 256
_GRID = _N // _BLK


def _mega_body(adj_ref, gn_ref, feat_ref, feata_ref, w1_ref, w2_ref, w0_ref,
               b_ref,
               z_ref, emb_ref, emba_ref, ret_ref, reta_ref, h_ref,
               adj8_s, dinv_s, xs_s, ys_s, embcat_s):
    p = pl.program_id(0)
    i = pl.program_id(1)
    f32 = jnp.float32
    bf16 = jnp.bfloat16

    @pl.when(p == 0)
    def _prep():
        a = adj_ref[...]
        row_ids = jax.lax.broadcasted_iota(jnp.int32, (_BLK, _N), 0) + i * _BLK
        col_ids = jax.lax.broadcasted_iota(jnp.int32, (_BLK, _N), 1)
        ondiag = (row_ids == col_ids) & (a == 0.0)
        a_sl = jnp.where(ondiag, 1.0, a)
        deg = jnp.sum(a_sl, axis=1)
        dinv = jnp.where(deg > 0.0, jax.lax.rsqrt(deg), 0.0)
        dinv_s[i] = jnp.broadcast_to(dinv[:, None], (_BLK, 128))
        adj8_s[i] = a_sl.astype(jnp.int8)

    @pl.when((p == 1) & (i == 0))
    def _xw():
        xw = jnp.dot(feat_ref[...], w1_ref[...], preferred_element_type=f32)
        xwa = jnp.dot(feata_ref[...], w1_ref[...], preferred_element_type=f32)
        dinvf = dinv_s[...].reshape(_N, 128)
        xs_s[...] = (jnp.concatenate([xw, xwa], axis=1) * dinvf).astype(bf16)

    @pl.when(p == 1)
    def _prop1():
        a8 = adj8_s[i].astype(bf16)
        acc = jnp.dot(a8, xs_s[...], preferred_element_type=f32)
        dinvb = dinv_s[i]
        zc = acc * dinvb
        z = zc[:, :64]
        za = zc[:, 64:]
        emb = jnp.maximum(z, 0.0)
        emba = jnp.maximum(za, 0.0)
        z_ref[...] = z
        emb_ref[...] = emb
        emba_ref[...] = emba
        embcat_s[i] = jnp.concatenate([emb, emba], axis=1).astype(bf16)
        ys = jnp.dot(z, w2_ref[...], preferred_element_type=f32) * dinvb
        ys_s[i] = ys.astype(bf16)

    @pl.when(p == 2)
    def _readout():
        g = gn_ref[...]
        vs = jnp.dot(g.astype(bf16), embcat_s[...].reshape(_N, 128),
                     preferred_element_type=f32)
        rs = jnp.sum(g, axis=1)
        gc = vs / rs[:, None]
        gp = gc[:, :64]
        gpa = gc[:, 64:]

        def l2sig(x):
            nrm = jnp.sqrt(jnp.sum(x * x, axis=1, keepdims=True))
            return jax.nn.sigmoid(x / jnp.maximum(nrm, 1e-12))

        gp = l2sig(gp)
        gpa = l2sig(gpa)
        ec = embcat_s[i].astype(f32)
        w0 = w0_ref[0]
        hw = jnp.dot(ec[:, :64], w0, preferred_element_type=f32)
        hwa = jnp.dot(ec[:, 64:], w0, preferred_element_type=f32)
        b = b_ref[0, 0]
        r0 = jnp.sum(hw * gp, axis=1, keepdims=True) + b
        r1 = jnp.sum(hwa * gp, axis=1, keepdims=True) + b
        ra0 = jnp.sum(hwa * gpa, axis=1, keepdims=True) + b
        ra1 = jnp.sum(hw * gpa, axis=1, keepdims=True) + b
        ret_ref[...] = jnp.concatenate([r0, r1], axis=1)
        reta_ref[...] = jnp.concatenate([ra0, ra1], axis=1)

    @pl.when(p == 3)
    def _prop2():
        a8 = adj8_s[i].astype(bf16)
        h_ref[...] = jnp.dot(a8, ys_s[...].reshape(_N, 128),
                             preferred_element_type=f32) * dinv_s[i]


def _owned(phase, width):
    def m(p, i):
        blk = jnp.where(p == phase, i,
                        jnp.where(p < phase, 0, _GRID - 1))
        return (blk, 0)
    del width
    return m


def _const(shape):
    nd = len(shape)
    return pl.BlockSpec(shape, lambda *_, _nd=nd: (0,) * _nd)


def kernel(feat, feat_a, adj, graph_neigh, W1, W2, disc_W, disc_b):
    f32 = jnp.float32
    bf16 = jnp.bfloat16

    z, emb, emb_a, ret, ret_a, h = pl.pallas_call(
        _mega_body,
        grid=(4, _GRID),
        in_specs=[
            pl.BlockSpec((_BLK, _N), _owned(0, _N)),
            pl.BlockSpec((_BLK, _N), _owned(2, _N)),
            _const((_N, 128)),
            _const((_N, 128)),
            _const((128, 64)),
            _const((64, 128)),
            _const((1, 64, 64)),
            _const((1, 1)),
        ],
        out_specs=[
            pl.BlockSpec((_BLK, 64), _owned(1, 64)),
            pl.BlockSpec((_BLK, 64), _owned(1, 64)),
            pl.BlockSpec((_BLK, 64), _owned(1, 64)),
            pl.BlockSpec((_BLK, 2), _owned(2, 2)),
            pl.BlockSpec((_BLK, 2), _owned(2, 2)),
            pl.BlockSpec((_BLK, 128), _owned(3, 128)),
        ],
        out_shape=[
            jax.ShapeDtypeStruct((_N, 64), f32),
            jax.ShapeDtypeStruct((_N, 64), f32),
            jax.ShapeDtypeStruct((_N, 64), f32),
            jax.ShapeDtypeStruct((_N, 2), f32),
            jax.ShapeDtypeStruct((_N, 2), f32),
            jax.ShapeDtypeStruct((_N, 128), f32),
        ],
        scratch_shapes=[
            pltpu.VMEM((_GRID, _BLK, _N), jnp.int8),
            pltpu.VMEM((_GRID, _BLK, 128), f32),
            pltpu.VMEM((_N, 128), bf16),
            pltpu.VMEM((_GRID, _BLK, 128), bf16),
            pltpu.VMEM((_GRID, _BLK, 128), bf16),
        ],
        compiler_params=pltpu.CompilerParams(
            vmem_limit_bytes=100 * 1024 * 1024,
        ),
    )(adj, graph_neigh, feat, feat_a, W1, W2, disc_W, disc_b.reshape(1, 1))

    return (z, h, ret, ret_a, emb, emb_a)
